# baseline (speedup 1.0000x reference)
import functools

import jax
import jax.numpy as jnp
from jax import lax
from jax.experimental import pallas as pl
from jax.experimental.pallas import tpu as pltpu
from jax.experimental.pallas import tpu_sc as plsc

VOCAB = 1000000
D = 64
SEQ = 200
BATCH = 4096

NC = 2
NS = 16
NW = NC * NS

SPW = BATCH // NW
NBUF = 4
LANES = 16
OFFS = (0, 56, 104, 152)
SIZES = (56, 48, 48, 48)
HMAX = SIZES[0]
CPW = 4 * SPW

_mesh = plsc.VectorSubcoreMesh(core_axis_name="c", subcore_axis_name="s")


@functools.partial(
    pl.kernel,
    mesh=_mesh,
    out_type=jax.ShapeDtypeStruct((BATCH, SEQ, 2 * D), jnp.float32),
    scratch_types=[
        pltpu.VMEM((SPW * SEQ,), jnp.int32),
        pltpu.VMEM((SPW * SEQ,), jnp.int32),
        pltpu.VMEM((SEQ * D,), jnp.float32),
        pltpu.VMEM((NBUF, HMAX, 2 * D), jnp.float32),
        pltpu.VMEM((NBUF, HMAX, 2 * D), jnp.float32),
        pltpu.SemaphoreType.DMA((NBUF,)),
        pltpu.SemaphoreType.DMA((NBUF,)),
    ],
)
def _gather(wide_hbm, idxw_hbm, lane_hbm, pos_hbm, out_hbm,
            idxw_v, lane_v, pos_v, rows_v, stg_v, gsem, osem):
    wid = lax.axis_index("s") * NC + lax.axis_index("c")
    seq0 = wid * SPW

    pltpu.make_async_copy(
        idxw_hbm.at[pl.ds(seq0 * SEQ, SPW * SEQ)], idxw_v, gsem.at[0]).start()
    pltpu.make_async_copy(
        lane_hbm.at[pl.ds(seq0 * SEQ, SPW * SEQ)], lane_v, gsem.at[1]).start()
    pltpu.make_async_copy(pos_hbm, pos_v, osem.at[0]).start()
    pltpu.make_async_copy(
        idxw_hbm.at[pl.ds(seq0 * SEQ, SPW * SEQ)], idxw_v, gsem.at[0]).wait()
    pltpu.make_async_copy(
        lane_hbm.at[pl.ds(seq0 * SEQ, SPW * SEQ)], lane_v, gsem.at[1]).wait()
    pltpu.make_async_copy(pos_hbm, pos_v, osem.at[0]).wait()

    def chunk_geom(b):
        return b, OFFS[b], SIZES[b]

    def start_gather(k, b):
        phase, off, size = chunk_geom(b)
        seq = (k - phase) // 4
        pltpu.make_async_copy(
            wide_hbm.at[idxw_v.at[pl.ds(seq * SEQ + off, size)]],
            rows_v.at[b, pl.ds(0, size)], gsem.at[b]).start()

    def wait_gather(k, b):
        phase, off, size = chunk_geom(b)
        seq = (k - phase) // 4
        pltpu.make_async_copy(
            wide_hbm.at[idxw_v.at[pl.ds(seq * SEQ + off, size)]],
            rows_v.at[b, pl.ds(0, size)], gsem.at[b]).wait()

    def start_out(k, b):
        phase, off, size = chunk_geom(b)
        seq = (k - phase) // 4
        pltpu.make_async_copy(
            stg_v.at[b, pl.ds(0, size)],
            out_hbm.at[seq0 + seq, pl.ds(off, size)],
            osem.at[b]).start()

    def wait_out(k, b):
        phase, off, size = chunk_geom(b)
        seq = (k - phase) // 4
        pltpu.make_async_copy(
            stg_v.at[b, pl.ds(0, size)],
            out_hbm.at[seq0 + seq, pl.ds(off, size)],
            osem.at[b]).wait()

    def add_pos(k, b):
        phase, off, size = chunk_geom(b)
        seq = (k - phase) // 4
        base = seq * SEQ + off

        def row_body(r, carry):
            lane = lane_v[pl.ds(base + r, 1)][0]
            for c in range(D // LANES):
                x = rows_v[b, r, pl.ds(lane + c * LANES, LANES)]
                p = pos_v[pl.ds((off + r) * D + c * LANES, LANES)]
                stg_v[b, r, pl.ds(c * LANES, LANES)] = x + p
            return carry

        lax.fori_loop(0, size, row_body, 0, unroll=4)

    def step(k, b, first_round):
        wait_gather(k, b)
        add_pos(k, b)
        start_out(k, b)
        pb = (b - 1) % NBUF
        if first_round:
            start_gather(k + NBUF - 1, pb)
            if b != 0:
                wait_out(k - 1, pb)
        else:
            @pl.when(k + NBUF - 1 < CPW)
            def _():
                start_gather(k + NBUF - 1, pb)
                wait_out(k - 1, pb)

    for b in range(NBUF - 1):
        start_gather(b, b)

    for b in range(NBUF):
        step(b, b, first_round=True)

    def outer(g, carry):
        for b in range(NBUF):
            step(g * NBUF + b, b, first_round=False)
        return carry

    lax.fori_loop(1, CPW // NBUF, outer, 0)

    for b in range(NBUF):
        wait_out(CPW - NBUF + b, b)


def kernel(encoded_words, embed_weight, pos_emb_weight):
    wide = embed_weight.reshape(VOCAB // 2, 2 * D)
    idx = encoded_words.astype(jnp.int32).reshape(BATCH * SEQ)
    idxw = idx >> 1
    lane = (idx & 1) << 6
    pos = pos_emb_weight[:SEQ].reshape(SEQ * D)
    out = _gather(wide, idxw, lane, pos)
    return out[:, :, :D]


# speedup vs baseline: 1.8014x; 1.8014x over previous
import functools

import jax
import jax.numpy as jnp
from jax import lax
from jax.experimental import pallas as pl
from jax.experimental.pallas import tpu as pltpu
from jax.experimental.pallas import tpu_sc as plsc

VOCAB = 1000000
D = 64
SEQ = 200
BATCH = 4096

NC = 2
NS = 16
NW = NC * NS

SPW = BATCH // NW
NBUF = 4
LANES = 16
H0 = 104
H1 = SEQ - H0

_mesh = plsc.VectorSubcoreMesh(core_axis_name="c", subcore_axis_name="s")


@functools.partial(
    pl.kernel,
    mesh=_mesh,
    compiler_params=pltpu.CompilerParams(use_tc_tiling_on_sc=False),
    out_type=jax.ShapeDtypeStruct((BATCH, SEQ, 2 * D), jnp.float32),
    scratch_types=[
        pltpu.VMEM((SPW, SEQ), jnp.int32),
        pltpu.VMEM((SEQ, D), jnp.float32),
        pltpu.VMEM((NBUF, SEQ, D), jnp.float32),
        pltpu.SemaphoreType.DMA((NBUF,)),
        pltpu.SemaphoreType.DMA((NBUF,)),
    ],
)
def _emb_kernel(table_hbm, idx_hbm, pos_hbm, out_hbm,
                idx_v, pos_v, rows_v, gsem, osem):
    wid = lax.axis_index("s") * NC + lax.axis_index("c")
    seq0 = wid * SPW

    pltpu.make_async_copy(
        idx_hbm.at[pl.ds(seq0, SPW)], idx_v, gsem.at[0]).start()
    pltpu.make_async_copy(
        pos_hbm.at[pl.ds(0, SEQ)], pos_v, osem.at[0]).start()
    pltpu.make_async_copy(
        idx_hbm.at[pl.ds(seq0, SPW)], idx_v, gsem.at[0]).wait()
    pltpu.make_async_copy(
        pos_hbm.at[pl.ds(0, SEQ)], pos_v, osem.at[0]).wait()

    def start_gather(j, b):
        pltpu.make_async_copy(
            table_hbm.at[idx_v.at[j, pl.ds(0, H0)]],
            rows_v.at[b, pl.ds(0, H0)], gsem.at[b]).start()
        pltpu.make_async_copy(
            table_hbm.at[idx_v.at[j, pl.ds(H0, H1)]],
            rows_v.at[b, pl.ds(H0, H1)], gsem.at[b]).start()

    def wait_gather(j, b):
        pltpu.make_async_copy(
            table_hbm.at[idx_v.at[j, pl.ds(0, H0)]],
            rows_v.at[b, pl.ds(0, H0)], gsem.at[b]).wait()
        pltpu.make_async_copy(
            table_hbm.at[idx_v.at[j, pl.ds(H0, H1)]],
            rows_v.at[b, pl.ds(H0, H1)], gsem.at[b]).wait()

    def start_out(j, b):
        pltpu.make_async_copy(
            rows_v.at[b], out_hbm.at[seq0 + j, slice(None), pl.ds(0, D)],
            osem.at[b]).start()

    def wait_out(j, b):
        pltpu.make_async_copy(
            rows_v.at[b], out_hbm.at[seq0 + j, slice(None), pl.ds(0, D)],
            osem.at[b]).wait()

    def add_pos(b):
        def row_body(r, carry):
            for c in range(D // LANES):
                sl = pl.ds(c * LANES, LANES)
                plsc.addupdate(rows_v.at[b, r, sl], pos_v[r, sl])
            return carry

        lax.fori_loop(0, SEQ, row_body, 0, unroll=4)

    def step(j, b, first_round):
        wait_gather(j, b)
        add_pos(b)
        start_out(j, b)
        pb = (b - 1) % NBUF
        if first_round:
            if b != 0:
                wait_out(j - 1, pb)
            start_gather(j + NBUF - 1, pb)
        else:
            @pl.when(j + NBUF - 1 < SPW)
            def _():
                wait_out(j - 1, pb)
                start_gather(j + NBUF - 1, pb)

    for b in range(NBUF - 1):
        start_gather(b, b)

    for b in range(NBUF):
        step(b, b, first_round=True)

    def outer(g, carry):
        for b in range(NBUF):
            step(g * NBUF + b, b, first_round=False)
        return carry

    lax.fori_loop(1, SPW // NBUF, outer, 0)

    for b in range(NBUF):
        wait_out(SPW - NBUF + b, b)


def kernel(encoded_words, embed_weight, pos_emb_weight):
    out = _emb_kernel(embed_weight,
                      encoded_words.astype(jnp.int32),
                      pos_emb_weight)
    return out[:, :, :D]
